# baseline (speedup 1.0000x reference)
import functools
import math

import jax
import jax.numpy as jnp
from jax import lax
from jax.experimental import pallas as pl
from jax.experimental.pallas import tpu as pltpu
from jax.experimental.pallas import tpu_sc as plsc

VOCAB = 1000000
D = 64
B = 16384
L = 50
NC, NS = 2, 16
NW = NC * NS
QW = B // (128 * NW)
NBLK = L * QW
NB = 4
SCALE = math.sqrt(D)


def _body(x_hbm, table_hbm, out_hbm, idx_v, rows_v, tiles_v, gsem, osem):
  c = lax.axis_index("c")
  s = lax.axis_index("s")
  wid = s * NC + c
  q0 = wid * QW

  pltpu.sync_copy(x_hbm.at[:, pl.ds(q0, QW)], idx_v)

  iot = lax.iota(jnp.int32, 16)
  pvec = [(iot + 16 * c) >> 3 for c in range(4)]
  dmvec = [(iot + 16 * c) & 7 for c in range(4)]
  bvec = [jnp.full((16,), b, jnp.int32) for b in range(NB)]

  def startg(i, b):
    l = i // QW
    qi = lax.rem(i, QW)
    pltpu.async_copy(
        table_hbm.at[idx_v.at[l, qi]], rows_v.at[b], gsem.at[b])

  def wait_g(b):
    pltpu.make_async_copy(
        table_hbm.at[pl.ds(0, 128)], rows_v.at[b], gsem.at[b]).wait()

  def start_o(i, b):
    l = i // QW
    qi = lax.rem(i, QW)
    pltpu.async_copy(
        tiles_v.at[b], out_hbm.at[l, :, q0 + qi], osem.at[b])

  def wait_o(b):
    pltpu.make_async_copy(
        tiles_v.at[b], out_hbm.at[0, :, 0], osem.at[b]).wait()

  def transpose_scale(b):
    @plsc.parallel_loop(0, 128, unroll=8)
    def _(r):
      rsplat = jnp.full((16,), r, jnp.int32)
      for c in range(4):
        v = rows_v[b, r, pl.ds(c * 16, 16)] * SCALE
        plsc.store_scatter(tiles_v, [bvec[b], pvec[c], dmvec[c], rsplat], v)

  startg(0, 0)
  startg(1, 1)

  @pl.loop(0, NBLK, step=NB)
  def _(i0):
    for bi in range(NB):
      i = i0 + bi
      b = bi
      b2 = (bi + 2) % NB
      wait_g(b)
      transpose_scale(b)

      @pl.when(i >= 2)
      def _():
        wait_o(b2)

      @pl.when(i + 2 < NBLK)
      def _():
        startg(i + 2, b2)

      start_o(i, b)

  wait_o((NBLK - 2) % NB)
  wait_o((NBLK - 1) % NB)


_emb = functools.partial(
    pl.kernel,
    out_type=jax.ShapeDtypeStruct((L, 8, B // 128, 8, 128), jnp.float32),
    mesh=plsc.VectorSubcoreMesh(core_axis_name="c", subcore_axis_name="s"),
    compiler_params=pltpu.CompilerParams(
        use_tc_tiling_on_sc=False, needs_layout_passes=False),
    scratch_types=[
        pltpu.VMEM((L, QW, 128), jnp.int32),
        pltpu.VMEM((NB, 128, D), jnp.float32),
        pltpu.VMEM((NB, 8, 8, 128), jnp.float32),
        pltpu.SemaphoreType.DMA((NB,)),
        pltpu.SemaphoreType.DMA((NB,)),
    ],
)(_body)


def kernel(x, table):
  xi = x.T.astype(jnp.int32).reshape(L, B // 128, 128)
  o5 = _emb(xi, table)
  return jnp.transpose(o5, (2, 4, 0, 1, 3)).reshape(B, L, D)


# speedup vs baseline: 1.6907x; 1.6907x over previous
import functools
import math

import jax
import jax.numpy as jnp
from jax import lax
from jax.experimental import pallas as pl
from jax.experimental.pallas import tpu as pltpu
from jax.experimental.pallas import tpu_sc as plsc

VOCAB = 1000000
D = 64
B = 16384
L = 50
NC, NS = 2, 16
NW = NC * NS
QW = B // (128 * NW)
NBLK = L * QW
NB = 4
SCALE = math.sqrt(D)


def _body(x_hbm, table_hbm, out_hbm, idx_v, rows_v, tiles_v, gsem, osem):
  c = lax.axis_index("c")
  s = lax.axis_index("s")
  wid = s * NC + c
  q0 = wid * QW

  pltpu.sync_copy(x_hbm.at[:, pl.ds(q0, QW)], idx_v)

  iot = lax.iota(jnp.int32, 16)
  pvec = [(iot + 16 * c) >> 3 for c in range(4)]
  dmvec = [(iot + 16 * c) & 7 for c in range(4)]
  bvec = [jnp.full((16,), b, jnp.int32) for b in range(NB)]

  def startg(i, b):
    l = i // QW
    qi = lax.rem(i, QW)
    pltpu.async_copy(
        table_hbm.at[idx_v.at[l, qi]], rows_v.at[b], gsem.at[b])

  def wait_g(b):
    pltpu.make_async_copy(
        table_hbm.at[pl.ds(0, 128)], rows_v.at[b], gsem.at[b]).wait()

  def start_o(i, b):
    l = i // QW
    qi = lax.rem(i, QW)
    pltpu.async_copy(
        tiles_v.at[b, :, :, pl.ds(0, 128)],
        out_hbm.at[l, :, q0 + qi], osem.at[b])

  def wait_o(b):
    pltpu.make_async_copy(
        tiles_v.at[b, :, :, pl.ds(0, 128)],
        out_hbm.at[0, :, 0], osem.at[b]).wait()

  def transpose_scale(b):
    @plsc.parallel_loop(0, 128, unroll=8)
    def _(r):
      rsplat = jnp.full((16,), r, jnp.int32)
      for c in range(4):
        v = rows_v[b, r, pl.ds(c * 16, 16)] * SCALE
        plsc.store_scatter(tiles_v, [bvec[b], pvec[c], dmvec[c], rsplat], v)

  startg(0, 0)
  startg(1, 1)

  @pl.loop(0, NBLK, step=NB)
  def _(i0):
    for bi in range(NB):
      i = i0 + bi
      b = bi
      b2 = (bi + 2) % NB
      wait_g(b)
      transpose_scale(b)

      @pl.when(i >= 2)
      def _():
        wait_o(b2)

      @pl.when(i + 2 < NBLK)
      def _():
        startg(i + 2, b2)

      start_o(i, b)

  wait_o((NBLK - 2) % NB)
  wait_o((NBLK - 1) % NB)


_emb = functools.partial(
    pl.kernel,
    out_type=jax.ShapeDtypeStruct((L, 8, B // 128, 8, 128), jnp.float32),
    mesh=plsc.VectorSubcoreMesh(core_axis_name="c", subcore_axis_name="s"),
    compiler_params=pltpu.CompilerParams(
        use_tc_tiling_on_sc=False, needs_layout_passes=False),
    scratch_types=[
        pltpu.VMEM((L, QW, 128), jnp.int32),
        pltpu.VMEM((NB, 128, D), jnp.float32),
        pltpu.VMEM((NB, 8, 8, 129), jnp.float32),
        pltpu.SemaphoreType.DMA((NB,)),
        pltpu.SemaphoreType.DMA((NB,)),
    ],
)(_body)


def kernel(x, table):
  xi = x.T.astype(jnp.int32).reshape(L, B // 128, 128)
  o5 = _emb(xi, table)
  return jnp.transpose(o5, (2, 4, 0, 1, 3)).reshape(B, L, D)
